# Initial kernel scaffold; baseline (speedup 1.0000x reference)
import jax
import jax.numpy as jnp
from jax.experimental import pallas as pl


def kernel(x_table, x_column, W1_tc_l, b1_tc_l, W1_tc_r, W1_ct_l, b1_ct_l, W1_ct_r, W2_tc_l, b2_tc_l, W2_tc_r, W2_ct_l, b2_ct_l, W2_ct_r, gn_w, gn_b, gn_ms, P1_w, P1_b, P2_w, P2_b, ei_tc, ei_ct):
    raise NotImplementedError("write your pallas kernel here")


# speedup vs baseline: 2.4828x; 2.4828x over previous
import functools

import jax
import jax.numpy as jnp
from jax import lax
from jax.experimental import pallas as pl
from jax.experimental.pallas import tpu as pltpu
from jax.experimental.pallas import tpu_sc as plsc

N = 5000
E = 160000
D = 128
NP = 5120
NC = 2
NS = 16
B = 128
K = (E + NC * NS * B - 1) // (NC * NS * B)
EPAD = NC * NS * K * B
RP = NP // NS

_sc_mesh = plsc.VectorSubcoreMesh(core_axis_name="c", subcore_axis_name="s")


@functools.partial(
    pl.kernel,
    out_type=[
        jax.ShapeDtypeStruct((NC, NP, D), jnp.float32),
        jax.ShapeDtypeStruct((NC, NP, D), jnp.float32),
    ],
    mesh=_sc_mesh,
    scratch_types=[
        pltpu.VMEM_SHARED((NP, D), jnp.float32),
        pltpu.VMEM_SHARED((NP, D), jnp.float32),
        pltpu.VMEM((K, B), jnp.int32),
        pltpu.VMEM((K, B), jnp.int32),
        pltpu.VMEM((B, D), jnp.float32),
        pltpu.VMEM((B, D), jnp.float32),
        pltpu.SemaphoreType.DMA,
    ],
)
def _sc_segsum_cnt(table_hbm, src_hbm, dst_hbm, z128_hbm, ones_hbm,
                   o_acc, o_cnt,
                   acc_sh, cnt_sh, idx_s_v, idx_d_v, rows_v, ones_v, sem):
    c = lax.axis_index("c")
    s = lax.axis_index("s")
    r0 = s * RP

    pltpu.sync_copy(z128_hbm.at[pl.ds(r0, RP)], acc_sh.at[pl.ds(r0, RP)])
    pltpu.sync_copy(z128_hbm.at[pl.ds(r0, RP)], cnt_sh.at[pl.ds(r0, RP)])
    pltpu.sync_copy(ones_hbm, ones_v)
    pltpu.sync_copy(src_hbm.at[c, s], idx_s_v)
    pltpu.sync_copy(dst_hbm.at[c, s], idx_d_v)
    plsc.subcore_barrier()

    def step(j, carry):
        pltpu.async_copy(table_hbm.at[idx_s_v.at[j]], rows_v, sem).wait()
        pltpu.sync_copy(rows_v, acc_sh.at[idx_d_v.at[j]], add=True)
        pltpu.sync_copy(ones_v, cnt_sh.at[idx_d_v.at[j]], add=True)
        return carry

    lax.fori_loop(0, K, step, 0)
    plsc.subcore_barrier()

    pltpu.sync_copy(acc_sh.at[pl.ds(r0, RP)], o_acc.at[c, pl.ds(r0, RP)])
    pltpu.sync_copy(cnt_sh.at[pl.ds(r0, RP)], o_cnt.at[c, pl.ds(r0, RP)])


@functools.partial(
    pl.kernel,
    out_type=[jax.ShapeDtypeStruct((NC, NP, D), jnp.float32)],
    mesh=_sc_mesh,
    scratch_types=[
        pltpu.VMEM_SHARED((NP, D), jnp.float32),
        pltpu.VMEM((K, B), jnp.int32),
        pltpu.VMEM((K, B), jnp.int32),
        pltpu.VMEM((B, D), jnp.float32),
        pltpu.SemaphoreType.DMA,
    ],
)
def _sc_layer2(h1c_hbm, sct_hbm, dct_hbm, z128_hbm,
               o_acc2, acc_sh, idx_s_v, idx_d_v, rows_v, sem):
    c = lax.axis_index("c")
    s = lax.axis_index("s")
    r0 = s * RP

    pltpu.sync_copy(z128_hbm.at[pl.ds(r0, RP)], acc_sh.at[pl.ds(r0, RP)])
    plsc.subcore_barrier()

    pltpu.sync_copy(sct_hbm.at[c, s], idx_s_v)
    pltpu.sync_copy(dct_hbm.at[c, s], idx_d_v)

    def step(j, carry):
        pltpu.async_copy(h1c_hbm.at[idx_s_v.at[j]], rows_v, sem).wait()
        pltpu.sync_copy(rows_v, acc_sh.at[idx_d_v.at[j]], add=True)
        return carry

    lax.fori_loop(0, K, step, 0)
    plsc.subcore_barrier()

    pltpu.sync_copy(acc_sh.at[pl.ds(r0, RP)], o_acc2.at[c, pl.ds(r0, RP)])


def _tc_layer1_body(acc_tc, cnt_tc, acc_ct, cnt_ct, xt, xc,
                    w1tl, b1tl, w1tr, w1cl, b1cl, w1cr,
                    h1c_o, h1t_o):
    f32 = jnp.float32

    s_tc = (acc_tc[0] + acc_tc[1])[:N]
    c_tc = (cnt_tc[0] + cnt_tc[1])[:N, 0:1]
    mean_tc = s_tc / jnp.maximum(c_tc, 1.0)
    h1c = (jnp.dot(mean_tc, w1tl[...], preferred_element_type=f32)
           + b1tl[...]
           + jnp.dot(xc[...], w1tr[...], preferred_element_type=f32))
    h1c_o[...] = jnp.maximum(h1c, 0.0)

    s_ct = (acc_ct[0] + acc_ct[1])[:N]
    c_ct = (cnt_ct[0] + cnt_ct[1])[:N, 0:1]
    mean_ct = s_ct / jnp.maximum(c_ct, 1.0)
    h1t = (jnp.dot(mean_ct, w1cl[...], preferred_element_type=f32)
           + b1cl[...]
           + jnp.dot(xt[...], w1cr[...], preferred_element_type=f32))
    h1t_o[...] = jnp.maximum(h1t, 0.0)


def _tc_layer2_body(acc2, cnt_ct, h1t,
                    w2cl, b2cl, w2cr, gn_w, gn_b, gn_ms,
                    p1w, p1b, p2w, p2b, out_o):
    f32 = jnp.float32

    s2 = (acc2[0] + acc2[1])[:N]
    c2 = (cnt_ct[0] + cnt_ct[1])[:N, 0:1]
    mean2 = s2 / jnp.maximum(c2, 1.0)
    x = (jnp.dot(mean2, w2cl[...], preferred_element_type=f32)
         + b2cl[...]
         + jnp.dot(h1t[...], w2cr[...], preferred_element_type=f32))

    mean0 = jnp.mean(x, axis=0, keepdims=True)
    ctr = x - gn_ms[...] * mean0
    var = jnp.mean(ctr * ctr, axis=0, keepdims=True)
    x = ctr * lax.rsqrt(var + 1e-5) * gn_w[...] + gn_b[...]

    x = jnp.maximum(jnp.dot(x, p1w[...], preferred_element_type=f32) + p1b[...], 0.0)
    x = jnp.dot(x, p2w[...], preferred_element_type=f32) + p2b[...]

    nrm = jnp.sqrt(jnp.sum(x * x, axis=1, keepdims=True))
    out_o[...] = x / jnp.maximum(nrm, 1e-12)


def _pad_edges(ei):
    src = ei[0].astype(jnp.int32)
    dst = ei[1].astype(jnp.int32)
    pad = EPAD - E
    src = jnp.concatenate([src, jnp.zeros((pad,), jnp.int32)])
    dst = jnp.concatenate([dst, jnp.full((pad,), N, jnp.int32)])
    return (src.reshape(NC, NS, K, B), dst.reshape(NC, NS, K, B))


def kernel(x_table, x_column, W1_tc_l, b1_tc_l, W1_tc_r, W1_ct_l, b1_ct_l,
           W1_ct_r, W2_tc_l, b2_tc_l, W2_tc_r, W2_ct_l, b2_ct_l, W2_ct_r,
           gn_w, gn_b, gn_ms, P1_w, P1_b, P2_w, P2_b, ei_tc, ei_ct):
    stc, dtc = _pad_edges(ei_tc)
    sct, dct = _pad_edges(ei_ct)
    z128 = jnp.zeros((NP, D), jnp.float32)
    ones = jnp.ones((B, D), jnp.float32)

    acc_tc, cnt_tc = _sc_segsum_cnt(x_table, stc, dtc, z128, ones)
    acc_ct, cnt_ct = _sc_segsum_cnt(x_column, sct, dct, z128, ones)

    row = lambda v: v.reshape(1, -1)
    h1_col, h1_tab = pl.pallas_call(
        _tc_layer1_body,
        out_shape=[jax.ShapeDtypeStruct((N, D), jnp.float32),
                   jax.ShapeDtypeStruct((N, D), jnp.float32)],
    )(acc_tc, cnt_tc, acc_ct, cnt_ct, x_table, x_column,
      W1_tc_l, row(b1_tc_l), W1_tc_r, W1_ct_l, row(b1_ct_l), W1_ct_r)

    (acc2,) = _sc_layer2(h1_col, sct, dct, z128)

    out = pl.pallas_call(
        _tc_layer2_body,
        out_shape=jax.ShapeDtypeStruct((N, D), jnp.float32),
    )(acc2, cnt_ct, h1_tab,
      W2_ct_l, row(b2_ct_l), W2_ct_r, row(gn_w), row(gn_b), row(gn_ms),
      P1_w, row(P1_b), P2_w, row(P2_b))
    return out
